# Initial kernel scaffold; baseline (speedup 1.0000x reference)
import jax
import jax.numpy as jnp
from jax.experimental import pallas as pl


def kernel(x, lut):
    raise NotImplementedError("write your pallas kernel here")


# speedup vs baseline: 1.2145x; 1.2145x over previous
import functools
import math

import jax
import jax.numpy as jnp
from jax import lax
from jax.experimental import pallas as pl
from jax.experimental.pallas import tpu as pltpu
from jax.experimental.pallas import tpu_sc as plsc

D_MODEL = 768
VOCAB = 100000
SCALE = math.sqrt(D_MODEL)

NC = 2
NS = 16
L = 16
NW = NC * NS

B = 4 * 8192
BPW = B // NW
C = 64
NCHUNK = BPW // C


def _emb_call(xf, lut):
    mesh = plsc.VectorSubcoreMesh(core_axis_name="c", subcore_axis_name="s")

    @functools.partial(
        pl.kernel,
        mesh=mesh,
        out_type=jax.ShapeDtypeStruct((B, D_MODEL), jnp.float32),
        scratch_types=[
            pltpu.VMEM((BPW,), jnp.int32),
            pltpu.VMEM((C, D_MODEL), jnp.float32),
            pltpu.SemaphoreType.DMA,
        ],
    )
    def k(idx_hbm, table_hbm, out_hbm, idx_v, rows_v, sem):
        wid = lax.axis_index("s") * NC + lax.axis_index("c")
        base = wid * BPW
        pltpu.sync_copy(idx_hbm.at[pl.ds(base, BPW)], idx_v)

        def chunk(ci, carry):
            off = ci * C
            pltpu.async_copy(
                table_hbm.at[idx_v.at[pl.ds(off, C)]], rows_v, sem
            ).wait()

            def srow(i, c2):
                for j in range(D_MODEL // L):
                    sl = pl.ds(j * L, L)
                    rows_v[i, sl] = rows_v[i, sl] * SCALE
                return c2

            lax.fori_loop(0, C, srow, 0)
            pltpu.sync_copy(rows_v, out_hbm.at[pl.ds(base + off, C)])
            return carry

        lax.fori_loop(0, NCHUNK, chunk, 0)

    return k(xf, lut)


def kernel(x, lut):
    xf = x.reshape(-1).astype(jnp.int32)
    out = _emb_call(xf, lut)
    return out.reshape(x.shape[0], x.shape[1], D_MODEL)


# speedup vs baseline: 1.6313x; 1.3432x over previous
import functools
import math

import jax
import jax.numpy as jnp
from jax import lax
from jax.experimental import pallas as pl
from jax.experimental.pallas import tpu as pltpu
from jax.experimental.pallas import tpu_sc as plsc

D_MODEL = 768
VOCAB = 100000
SCALE = math.sqrt(D_MODEL)

NC = 2
NS = 16
L = 16
NW = NC * NS

B = 4 * 8192
BPW = B // NW
C = 32
NCHUNK = BPW // C
NBUF = 4
GD = 2


def _emb_call(xf, lut):
    mesh = plsc.VectorSubcoreMesh(core_axis_name="c", subcore_axis_name="s")

    @functools.partial(
        pl.kernel,
        mesh=mesh,
        out_type=jax.ShapeDtypeStruct((B, D_MODEL), jnp.float32),
        scratch_types=(
            [pltpu.VMEM((BPW,), jnp.int32)]
            + [pltpu.VMEM((C, D_MODEL), jnp.float32)] * NBUF
            + [pltpu.SemaphoreType.DMA] * (2 * NBUF)
        ),
    )
    def k(idx_hbm, table_hbm, out_hbm, idx_v, *bufs_sems):
        bufs = bufs_sems[:NBUF]
        gsems = bufs_sems[NBUF:2 * NBUF]
        osems = bufs_sems[2 * NBUF:]
        wid = lax.axis_index("s") * NC + lax.axis_index("c")
        base = wid * BPW
        pltpu.sync_copy(idx_hbm.at[pl.ds(base, BPW)], idx_v)

        def gather_start(ci, b):
            pltpu.async_copy(
                table_hbm.at[idx_v.at[pl.ds(ci * C, C)]], bufs[b], gsems[b]
            )

        def gather_wait(b):
            pltpu.make_async_copy(
                table_hbm.at[idx_v.at[pl.ds(0, C)]], bufs[b], gsems[b]
            ).wait()

        def out_start(ci, b):
            pltpu.async_copy(
                bufs[b], out_hbm.at[pl.ds(base + ci * C, C)], osems[b]
            )

        def out_wait(b):
            pltpu.make_async_copy(
                bufs[b], out_hbm.at[pl.ds(base, C)], osems[b]
            ).wait()

        def scale(b):
            def srow(i, c2):
                for j in range(D_MODEL // L):
                    sl = pl.ds(j * L, L)
                    bufs[b][i, sl] = bufs[b][i, sl] * SCALE
                return c2

            lax.fori_loop(0, C, srow, 0)

        for ci in range(GD):
            gather_start(ci, ci % NBUF)

        def body(g, carry):
            for b in range(NBUF):
                ci = g * NBUF + b
                gather_wait(b)
                scale(b)
                out_start(ci, b)
                nb = (b + GD) % NBUF
                nxt = ci + GD

                @pl.when(nxt >= NBUF)
                def _():
                    out_wait(nb)

                @pl.when(nxt < NCHUNK)
                def _():
                    gather_start(nxt, nb)

            return carry

        lax.fori_loop(0, NCHUNK // NBUF, body, 0)

        for m in range(NCHUNK - (NBUF - GD), NCHUNK):
            out_wait(m % NBUF)

    return k(xf, lut)


def kernel(x, lut):
    xf = x.reshape(-1).astype(jnp.int32)
    out = _emb_call(xf, lut)
    return out.reshape(x.shape[0], x.shape[1], D_MODEL)


# speedup vs baseline: 1.6600x; 1.0176x over previous
import functools
import math

import jax
import jax.numpy as jnp
from jax import lax
from jax.experimental import pallas as pl
from jax.experimental.pallas import tpu as pltpu
from jax.experimental.pallas import tpu_sc as plsc

D_MODEL = 768
VOCAB = 100000
SCALE = math.sqrt(D_MODEL)

NC = 2
NS = 16
L = 16
NW = NC * NS

B = 4 * 8192
BPW = B // NW
C = 16
NCHUNK = BPW // C
NBUF = 8
GD = 4


def _emb_call(xf, lut):
    mesh = plsc.VectorSubcoreMesh(core_axis_name="c", subcore_axis_name="s")

    @functools.partial(
        pl.kernel,
        mesh=mesh,
        out_type=jax.ShapeDtypeStruct((B, D_MODEL), jnp.float32),
        scratch_types=(
            [pltpu.VMEM((BPW,), jnp.int32)]
            + [pltpu.VMEM((C, D_MODEL), jnp.float32)] * NBUF
            + [pltpu.SemaphoreType.DMA] * (2 * NBUF)
        ),
    )
    def k(idx_hbm, table_hbm, out_hbm, idx_v, *bufs_sems):
        bufs = bufs_sems[:NBUF]
        gsems = bufs_sems[NBUF:2 * NBUF]
        osems = bufs_sems[2 * NBUF:]
        wid = lax.axis_index("s") * NC + lax.axis_index("c")
        base = wid * BPW
        pltpu.sync_copy(idx_hbm.at[pl.ds(base, BPW)], idx_v)

        def gather_start(ci, b):
            pltpu.async_copy(
                table_hbm.at[idx_v.at[pl.ds(ci * C, C)]], bufs[b], gsems[b]
            )

        def gather_wait(b):
            pltpu.make_async_copy(
                table_hbm.at[idx_v.at[pl.ds(0, C)]], bufs[b], gsems[b]
            ).wait()

        def out_start(ci, b):
            pltpu.async_copy(
                bufs[b], out_hbm.at[pl.ds(base + ci * C, C)], osems[b]
            )

        def out_wait(b):
            pltpu.make_async_copy(
                bufs[b], out_hbm.at[pl.ds(base, C)], osems[b]
            ).wait()

        def scale(b):
            def srow(i, c2):
                for j in range(D_MODEL // L):
                    sl = pl.ds(j * L, L)
                    bufs[b][i, sl] = bufs[b][i, sl] * SCALE
                return c2

            lax.fori_loop(0, C, srow, 0)

        for ci in range(GD):
            gather_start(ci, ci % NBUF)

        def body(g, carry):
            for b in range(NBUF):
                ci = g * NBUF + b
                gather_wait(b)
                scale(b)
                out_start(ci, b)
                nb = (b + GD) % NBUF
                nxt = ci + GD

                @pl.when(nxt >= NBUF)
                def _():
                    out_wait(nb)

                @pl.when(nxt < NCHUNK)
                def _():
                    gather_start(nxt, nb)

            return carry

        lax.fori_loop(0, NCHUNK // NBUF, body, 0)

        for m in range(NCHUNK - (NBUF - GD), NCHUNK):
            out_wait(m % NBUF)

    return k(xf, lut)


def kernel(x, lut):
    xf = x.reshape(-1).astype(jnp.int32)
    out = _emb_call(xf, lut)
    return out.reshape(x.shape[0], x.shape[1], D_MODEL)


# speedup vs baseline: 1.6730x; 1.0078x over previous
import functools
import math

import jax
import jax.numpy as jnp
from jax import lax
from jax.experimental import pallas as pl
from jax.experimental.pallas import tpu as pltpu
from jax.experimental.pallas import tpu_sc as plsc

D_MODEL = 768
VOCAB = 100000
SCALE = math.sqrt(D_MODEL)

NC = 2
NS = 16
L = 16
NW = NC * NS

B = 4 * 8192
BPW = B // NW
C = 16
NCHUNK = BPW // C
NBUF = 8
GD = 6


def _emb_call(xf, lut):
    mesh = plsc.VectorSubcoreMesh(core_axis_name="c", subcore_axis_name="s")

    @functools.partial(
        pl.kernel,
        mesh=mesh,
        out_type=jax.ShapeDtypeStruct((B, D_MODEL), jnp.float32),
        scratch_types=(
            [pltpu.VMEM((BPW,), jnp.int32)]
            + [pltpu.VMEM((C, D_MODEL), jnp.float32)] * NBUF
            + [pltpu.SemaphoreType.DMA] * (2 * NBUF)
        ),
    )
    def k(idx_hbm, table_hbm, out_hbm, idx_v, *bufs_sems):
        bufs = bufs_sems[:NBUF]
        gsems = bufs_sems[NBUF:2 * NBUF]
        osems = bufs_sems[2 * NBUF:]
        wid = lax.axis_index("s") * NC + lax.axis_index("c")
        base = wid * BPW
        pltpu.sync_copy(idx_hbm.at[pl.ds(base, BPW)], idx_v)

        def gather_start(ci, b):
            pltpu.async_copy(
                table_hbm.at[idx_v.at[pl.ds(ci * C, C)]], bufs[b], gsems[b]
            )

        def gather_wait(b):
            pltpu.make_async_copy(
                table_hbm.at[idx_v.at[pl.ds(0, C)]], bufs[b], gsems[b]
            ).wait()

        def out_start(ci, b):
            pltpu.async_copy(
                bufs[b], out_hbm.at[pl.ds(base + ci * C, C)], osems[b]
            )

        def out_wait(b):
            pltpu.make_async_copy(
                bufs[b], out_hbm.at[pl.ds(base, C)], osems[b]
            ).wait()

        def scale(b):
            def srow(i, c2):
                for j in range(D_MODEL // L):
                    sl = pl.ds(j * L, L)
                    bufs[b][i, sl] = bufs[b][i, sl] * SCALE
                return c2

            lax.fori_loop(0, C, srow, 0)

        for ci in range(GD):
            gather_start(ci, ci % NBUF)

        def body(g, carry):
            for b in range(NBUF):
                ci = g * NBUF + b
                gather_wait(b)
                scale(b)
                out_start(ci, b)
                nb = (b + GD) % NBUF
                nxt = ci + GD

                @pl.when(nxt >= NBUF)
                def _():
                    out_wait(nb)

                @pl.when(nxt < NCHUNK)
                def _():
                    gather_start(nxt, nb)

            return carry

        lax.fori_loop(0, NCHUNK // NBUF, body, 0)

        for m in range(NCHUNK - (NBUF - GD), NCHUNK):
            out_wait(m % NBUF)

    return k(xf, lut)


def kernel(x, lut):
    xf = x.reshape(-1).astype(jnp.int32)
    out = _emb_call(xf, lut)
    return out.reshape(x.shape[0], x.shape[1], D_MODEL)


# speedup vs baseline: 1.6732x; 1.0001x over previous
import functools
import math

import jax
import jax.numpy as jnp
from jax import lax
from jax.experimental import pallas as pl
from jax.experimental.pallas import tpu as pltpu
from jax.experimental.pallas import tpu_sc as plsc

D_MODEL = 768
VOCAB = 100000
SCALE = math.sqrt(D_MODEL)

NC = 2
NS = 16
L = 16
NW = NC * NS

B = 4 * 8192
BPW = B // NW
C = 16
NCHUNK = BPW // C
NBUF = 8
GD = 5


def _emb_call(xf, lut):
    mesh = plsc.VectorSubcoreMesh(core_axis_name="c", subcore_axis_name="s")

    @functools.partial(
        pl.kernel,
        mesh=mesh,
        out_type=jax.ShapeDtypeStruct((B, D_MODEL), jnp.float32),
        scratch_types=(
            [pltpu.VMEM((BPW,), jnp.int32)]
            + [pltpu.VMEM((C, D_MODEL), jnp.float32)] * NBUF
            + [pltpu.SemaphoreType.DMA] * (2 * NBUF)
        ),
    )
    def k(idx_hbm, table_hbm, out_hbm, idx_v, *bufs_sems):
        bufs = bufs_sems[:NBUF]
        gsems = bufs_sems[NBUF:2 * NBUF]
        osems = bufs_sems[2 * NBUF:]
        wid = lax.axis_index("s") * NC + lax.axis_index("c")
        base = wid * BPW
        pltpu.sync_copy(idx_hbm.at[pl.ds(base, BPW)], idx_v)

        def gather_start(ci, b):
            pltpu.async_copy(
                table_hbm.at[idx_v.at[pl.ds(ci * C, C)]], bufs[b], gsems[b]
            )

        def gather_wait(b):
            pltpu.make_async_copy(
                table_hbm.at[idx_v.at[pl.ds(0, C)]], bufs[b], gsems[b]
            ).wait()

        def out_start(ci, b):
            pltpu.async_copy(
                bufs[b], out_hbm.at[pl.ds(base + ci * C, C)], osems[b]
            )

        def out_wait(b):
            pltpu.make_async_copy(
                bufs[b], out_hbm.at[pl.ds(base, C)], osems[b]
            ).wait()

        def scale(b):
            def srow(i, c2):
                for j in range(D_MODEL // L):
                    sl = pl.ds(j * L, L)
                    bufs[b][i, sl] = bufs[b][i, sl] * SCALE
                return c2

            lax.fori_loop(0, C, srow, 0)

        for ci in range(GD):
            gather_start(ci, ci % NBUF)

        def body(g, carry):
            for b in range(NBUF):
                ci = g * NBUF + b
                gather_wait(b)
                scale(b)
                out_start(ci, b)
                nb = (b + GD) % NBUF
                nxt = ci + GD

                @pl.when(nxt >= NBUF)
                def _():
                    out_wait(nb)

                @pl.when(nxt < NCHUNK)
                def _():
                    gather_start(nxt, nb)

            return carry

        lax.fori_loop(0, NCHUNK // NBUF, body, 0)

        for m in range(NCHUNK - (NBUF - GD), NCHUNK):
            out_wait(m % NBUF)

    return k(xf, lut)


def kernel(x, lut):
    xf = x.reshape(-1).astype(jnp.int32)
    out = _emb_call(xf, lut)
    return out.reshape(x.shape[0], x.shape[1], D_MODEL)
